# Initial kernel scaffold; baseline (speedup 1.0000x reference)
import jax
import jax.numpy as jnp
from jax.experimental import pallas as pl


def kernel(x, embeddings, count):
    raise NotImplementedError("write your pallas kernel here")


# speedup vs baseline: 3.4013x; 3.4013x over previous
import jax
import jax.numpy as jnp
from jax.experimental import pallas as pl
from jax.experimental.pallas import tpu as pltpu

BN = 2048


def _vq_body(x_ref, et_ref, cnt_ref, xq_ref, idx_ref, ncnt_ref):
    nb = pl.program_id(0)
    G = et_ref.shape[0]
    D = et_ref.shape[1]
    K = et_ref.shape[2]
    idx_cols = []
    xq_cols = []
    cnt_rows = []
    for g in range(G):
        xg = x_ref[:, g * D:(g + 1) * D]
        et = et_ref[g]
        x_sq = jnp.sum(xg * xg, axis=-1, keepdims=True)
        e_sq = jnp.sum(et * et, axis=0, keepdims=True)
        cross = jnp.dot(xg, et, preferred_element_type=jnp.float32)
        d2 = jnp.maximum((x_sq - 2.0 * cross) + e_sq, 0.0)
        m = jnp.min(d2, axis=-1, keepdims=True)
        kiota = jax.lax.broadcasted_iota(jnp.int32, d2.shape, 1)
        idxc = jnp.min(jnp.where(d2 == m, kiota, K), axis=-1, keepdims=True)
        onehot = (kiota == idxc).astype(jnp.float32)
        xq = jax.lax.dot_general(onehot, et, (((1,), (1,)), ((), ())),
                                 preferred_element_type=jnp.float32)
        idx_cols.append(idxc)
        xq_cols.append(xq)
        cnt_rows.append(jnp.sum(onehot, axis=0, keepdims=True))
    idx_ref[...] = jnp.concatenate(idx_cols, axis=1)
    xq_ref[...] = jnp.concatenate(xq_cols, axis=1)
    contrib = jnp.concatenate(cnt_rows, axis=0)

    @pl.when(nb == 0)
    def _():
        ncnt_ref[...] = cnt_ref[...] + contrib

    @pl.when(nb > 0)
    def _():
        ncnt_ref[...] = ncnt_ref[...] + contrib


def kernel(x, embeddings, count):
    BS, TPD, D = x.shape
    G, K, _ = embeddings.shape
    N = BS * TPD // G
    x2d = x.reshape(N, G * D)
    e_t = embeddings.transpose(0, 2, 1)
    grid = (N // BN,)
    xq2d, idx_all, ncnt = pl.pallas_call(
        _vq_body,
        grid=grid,
        in_specs=[
            pl.BlockSpec((BN, G * D), lambda i: (i, 0)),
            pl.BlockSpec((G, D, K), lambda i: (0, 0, 0)),
            pl.BlockSpec((G, K), lambda i: (0, 0)),
        ],
        out_specs=[
            pl.BlockSpec((BN, G * D), lambda i: (i, 0)),
            pl.BlockSpec((BN, G), lambda i: (i, 0)),
            pl.BlockSpec((G, K), lambda i: (0, 0)),
        ],
        out_shape=[
            jax.ShapeDtypeStruct((N, G * D), jnp.float32),
            jax.ShapeDtypeStruct((N, G), jnp.int32),
            jax.ShapeDtypeStruct((G, K), jnp.float32),
        ],
        compiler_params=pltpu.CompilerParams(
            dimension_semantics=("arbitrary",)),
    )(x2d, e_t, count)
    return xq2d.reshape(BS, TPD, D), idx_all, ncnt
